# Initial kernel scaffold; baseline (speedup 1.0000x reference)
import jax
import jax.numpy as jnp
from jax.experimental import pallas as pl


def kernel(id1, W):
    raise NotImplementedError("write your pallas kernel here")


# speedup vs baseline: 4.8957x; 4.8957x over previous
import functools

import jax
import jax.numpy as jnp
from jax import lax
from jax.experimental import pallas as pl
from jax.experimental.pallas import tpu as pltpu
from jax.experimental.pallas import tpu_sc as plsc

NC = 2
NS = 16
NW = NC * NS
L = 16
EMB = 5

N_ROWS, N_COLS = 16384, 200
N = N_ROWS * N_COLS
PER_W = N // NW
CHUNK = 12800
N_CHUNKS = PER_W // CHUNK


def _sc_body(id_hbm, w_hbm, out_hbm, id_v, out_v, w_v):
    wid = lax.axis_index("s") * NC + lax.axis_index("c")
    pltpu.sync_copy(w_hbm, w_v)

    iota = lax.iota(jnp.int32, L)
    p0, p1, gidx = [], [], []
    for v in range(EMB):
        pos = iota + L * v
        k_mod = pos % EMB
        p0.append(plsc.load_gather(w_v, [k_mod]))
        p1.append(plsc.load_gather(w_v, [k_mod + EMB]))
        gidx.append(pos // EMB)

    base_w = wid * PER_W
    for c in range(N_CHUNKS):
        base = base_w + c * CHUNK
        pltpu.sync_copy(id_hbm.at[pl.ds(base, CHUNK)], id_v)

        def body(t, carry):
            b16 = t * L
            for v in range(EMB):
                g = plsc.load_gather(id_v, [gidx[v] + b16])
                o = jnp.where(g == 0, p0[v], p1[v])
                out_v[pl.ds(t * (L * EMB) + v * L, L)] = o
            return carry

        lax.fori_loop(0, CHUNK // L, body, 0)
        pltpu.sync_copy(out_v, out_hbm.at[pl.ds(base * EMB, CHUNK * EMB)])


_mesh = plsc.VectorSubcoreMesh(core_axis_name="c", subcore_axis_name="s")

_sc_lookup = functools.partial(
    pl.kernel,
    mesh=_mesh,
    out_type=jax.ShapeDtypeStruct((N * EMB,), jnp.float32),
    scratch_types=[
        pltpu.VMEM((CHUNK,), jnp.int32),
        pltpu.VMEM((CHUNK * EMB,), jnp.float32),
        pltpu.VMEM((L,), jnp.float32),
    ],
    compiler_params=pltpu.CompilerParams(needs_layout_passes=False),
)(_sc_body)


def kernel(id1, W):
    ids = id1.reshape(-1).astype(jnp.int32)
    wflat = jnp.pad(W.reshape(-1), (0, L - 2 * EMB)).astype(jnp.float32)
    out = _sc_lookup(ids, wflat)
    return out.reshape(N_ROWS, N_COLS, EMB)


# speedup vs baseline: 5.1552x; 1.0530x over previous
import functools

import jax
import jax.numpy as jnp
from jax import lax
from jax.experimental import pallas as pl
from jax.experimental.pallas import tpu as pltpu
from jax.experimental.pallas import tpu_sc as plsc

NC = 2
NS = 16
NW = NC * NS
L = 16
EMB = 5

N_ROWS, N_COLS = 16384, 200
N = N_ROWS * N_COLS
PER_W = N // NW
CHUNK = 12800
N_CHUNKS = PER_W // CHUNK


def _sc_body(id_hbm, w_hbm, out_hbm, id_v, out_v, w_v):
    wid = lax.axis_index("s") * NC + lax.axis_index("c")
    pltpu.sync_copy(w_hbm, w_v)

    iota = lax.iota(jnp.int32, L)
    p0, p1, gidx = [], [], []
    for v in range(EMB):
        pos = iota + L * v
        k_mod = pos % EMB
        p0.append(plsc.load_gather(w_v, [k_mod]))
        p1.append(plsc.load_gather(w_v, [k_mod + EMB]))
        gidx.append(pos // EMB)

    base_w = wid * PER_W
    for c in range(N_CHUNKS):
        base = base_w + c * CHUNK
        pltpu.sync_copy(id_hbm.at[pl.ds(base, CHUNK)], id_v)

        @plsc.parallel_loop(0, CHUNK // L, unroll=8)
        def body(t):
            b16 = t * L
            for v in range(EMB):
                g = plsc.load_gather(id_v, [gidx[v] + b16])
                o = jnp.where(g == 0, p0[v], p1[v])
                out_v[pl.ds(t * (L * EMB) + v * L, L)] = o
        pltpu.sync_copy(out_v, out_hbm.at[pl.ds(base * EMB, CHUNK * EMB)])


_mesh = plsc.VectorSubcoreMesh(core_axis_name="c", subcore_axis_name="s")

_sc_lookup = functools.partial(
    pl.kernel,
    mesh=_mesh,
    out_type=jax.ShapeDtypeStruct((N * EMB,), jnp.float32),
    scratch_types=[
        pltpu.VMEM((CHUNK,), jnp.int32),
        pltpu.VMEM((CHUNK * EMB,), jnp.float32),
        pltpu.VMEM((L,), jnp.float32),
    ],
    compiler_params=pltpu.CompilerParams(needs_layout_passes=False),
)(_sc_body)


def kernel(id1, W):
    ids = id1.reshape(-1).astype(jnp.int32)
    wflat = jnp.pad(W.reshape(-1), (0, L - 2 * EMB)).astype(jnp.float32)
    out = _sc_lookup(ids, wflat)
    return out.reshape(N_ROWS, N_COLS, EMB)


# speedup vs baseline: 146.0307x; 28.3266x over previous
import functools

import jax
import jax.numpy as jnp
from jax import lax
from jax.experimental import pallas as pl
from jax.experimental.pallas import tpu as pltpu
from jax.experimental.pallas import tpu_sc as plsc

NC = 2
NS = 16
NW = NC * NS
L = 16
EMB = 5

N_ROWS, N_COLS = 16384, 200
I_SPAN = N_ROWS // NW
NJT = N_COLS // 8


def _sc_body(idt_hbm, w_hbm, out_hbm, id_v, out_v, w_v):
    wid = lax.axis_index("s") * NC + lax.axis_index("c")
    pltpu.sync_copy(w_hbm, w_v)
    p0 = [plsc.load_gather(w_v, [jnp.full((L,), k, jnp.int32)])
          for k in range(EMB)]
    p1 = [plsc.load_gather(w_v, [jnp.full((L,), k + EMB, jnp.int32)])
          for k in range(EMB)]
    i0 = wid * I_SPAN

    def jt_body(jt, carry):
        j0 = jt * 8
        pltpu.sync_copy(idt_hbm.at[pl.ds(j0, 8), pl.ds(i0, I_SPAN)], id_v)

        @plsc.parallel_loop(0, I_SPAN // L, unroll=2)
        def c_body(c):
            off = c * L
            for j in range(8):
                m = id_v[j, pl.ds(off, L)] == 0
                for k in range(EMB):
                    out_v[k, j, pl.ds(off, L)] = jnp.where(m, p0[k], p1[k])

        for k in range(EMB):
            pltpu.sync_copy(out_v.at[k],
                            out_hbm.at[k, pl.ds(j0, 8), pl.ds(i0, I_SPAN)])
        return carry

    lax.fori_loop(0, NJT, jt_body, 0)


_mesh = plsc.VectorSubcoreMesh(core_axis_name="c", subcore_axis_name="s")

_sc_lookup = functools.partial(
    pl.kernel,
    mesh=_mesh,
    out_type=jax.ShapeDtypeStruct((EMB, N_COLS, N_ROWS), jnp.float32),
    scratch_types=[
        pltpu.VMEM((8, I_SPAN), jnp.int32),
        pltpu.VMEM((EMB, 8, I_SPAN), jnp.float32),
        pltpu.VMEM((L,), jnp.float32),
    ],
    compiler_params=pltpu.CompilerParams(needs_layout_passes=False),
)(_sc_body)


def kernel(id1, W):
    idt = id1.T
    wflat = jnp.pad(W.reshape(-1), (0, L - 2 * EMB)).astype(jnp.float32)
    out_t = _sc_lookup(idt, wflat)
    return jnp.transpose(out_t, (2, 1, 0))


# speedup vs baseline: 146.0852x; 1.0004x over previous
import functools

import jax
import jax.numpy as jnp
from jax import lax
from jax.experimental import pallas as pl
from jax.experimental.pallas import tpu as pltpu
from jax.experimental.pallas import tpu_sc as plsc

NC = 2
NS = 16
NW = NC * NS
L = 16
EMB = 5

N_ROWS, N_COLS = 16384, 200
I_SPAN = N_ROWS // NW
NJT = N_COLS // 8


def _sc_body(idt_hbm, w_hbm, out_hbm, id_v, out_v, w_v):
    wid = lax.axis_index("s") * NC + lax.axis_index("c")
    pltpu.sync_copy(w_hbm, w_v)
    p0 = [plsc.load_gather(w_v, [jnp.full((L,), k + 1, jnp.int32)])
          for k in range(EMB)]
    p1 = [plsc.load_gather(w_v, [jnp.full((L,), k + 1 + EMB, jnp.int32)])
          for k in range(EMB)]
    i0 = wid * I_SPAN

    def jt_body(jt, carry):
        j0 = jt * 8
        pltpu.sync_copy(idt_hbm.at[pl.ds(j0, 8), pl.ds(i0, I_SPAN)], id_v)

        @plsc.parallel_loop(0, I_SPAN // L, unroll=2)
        def c_body(c):
            off = c * L
            for j in range(8):
                m = id_v[j, pl.ds(off, L)] == 0
                for k in range(EMB):
                    out_v[k, j, pl.ds(off, L)] = jnp.where(m, p0[k], p1[k])

        for k in range(EMB):
            pltpu.sync_copy(out_v.at[k],
                            out_hbm.at[k, pl.ds(j0, 8), pl.ds(i0, I_SPAN)])
        return carry

    lax.fori_loop(0, NJT, jt_body, 0)


_mesh = plsc.VectorSubcoreMesh(core_axis_name="c", subcore_axis_name="s")

_sc_lookup = functools.partial(
    pl.kernel,
    mesh=_mesh,
    out_type=jax.ShapeDtypeStruct((EMB, N_COLS, N_ROWS), jnp.float32),
    scratch_types=[
        pltpu.VMEM((8, I_SPAN), jnp.int32),
        pltpu.VMEM((EMB, 8, I_SPAN), jnp.float32),
        pltpu.VMEM((L,), jnp.float32),
    ],
    compiler_params=pltpu.CompilerParams(needs_layout_passes=False),
)(_sc_body)


def kernel(id1, W):
    idt = id1.T
    wflat = jnp.pad(W.reshape(-1), (1, L - 2 * EMB - 1)).astype(jnp.float32)
    out_t = _sc_lookup(idt, wflat)
    return jnp.transpose(out_t, (2, 1, 0))


# speedup vs baseline: 204.7880x; 1.4018x over previous
import functools

import jax
import jax.numpy as jnp
from jax import lax
from jax.experimental import pallas as pl
from jax.experimental.pallas import tpu as pltpu
from jax.experimental.pallas import tpu_sc as plsc

NC = 2
NS = 16
NW = NC * NS
L = 16
EMB = 5

N_ROWS, N_COLS = 16384, 200
I_SPAN = N_ROWS // NW
NJT = N_COLS // 8


def _sc_body(idt_hbm, w_hbm, out_hbm, id_v, out_v, w_v,
             in_sem0, in_sem1, out_sem0, out_sem1):
    wid = lax.axis_index("s") * NC + lax.axis_index("c")
    pltpu.sync_copy(w_hbm, w_v)
    p0 = [plsc.load_gather(w_v, [jnp.full((L,), k + 1, jnp.int32)])
          for k in range(EMB)]
    p1 = [plsc.load_gather(w_v, [jnp.full((L,), k + 1 + EMB, jnp.int32)])
          for k in range(EMB)]
    i0 = wid * I_SPAN

    in_sems = [in_sem0, in_sem1]
    out_sems = [out_sem0, out_sem1]
    pend_in = [None, None]
    pend_out = [[], []]

    def start_in(jt):
        b = jt & 1
        pend_in[b] = pltpu.async_copy(
            idt_hbm.at[pl.ds(jt * 8, 8), pl.ds(i0, I_SPAN)],
            id_v.at[b], in_sems[b])

    start_in(0)
    for jt in range(NJT):
        b = jt & 1
        if jt + 1 < NJT:
            start_in(jt + 1)
        pend_in[b].wait()
        for h in pend_out[b]:
            h.wait()
        pend_out[b] = []

        @plsc.parallel_loop(0, I_SPAN // L, unroll=2)
        def c_body(c):
            off = c * L
            for j in range(8):
                m = id_v[b, j, pl.ds(off, L)] == 0
                for k in range(EMB):
                    out_v[b, k, j, pl.ds(off, L)] = jnp.where(m, p0[k], p1[k])

        for k in range(EMB):
            pend_out[b].append(pltpu.async_copy(
                out_v.at[b, k],
                out_hbm.at[k, pl.ds(jt * 8, 8), pl.ds(i0, I_SPAN)],
                out_sems[b]))

    for b in range(2):
        for h in pend_out[b]:
            h.wait()


_mesh = plsc.VectorSubcoreMesh(core_axis_name="c", subcore_axis_name="s")

_sc_lookup = functools.partial(
    pl.kernel,
    mesh=_mesh,
    out_type=jax.ShapeDtypeStruct((EMB, N_COLS, N_ROWS), jnp.float32),
    scratch_types=[
        pltpu.VMEM((2, 8, I_SPAN), jnp.int32),
        pltpu.VMEM((2, EMB, 8, I_SPAN), jnp.float32),
        pltpu.VMEM((L,), jnp.float32),
        pltpu.SemaphoreType.DMA,
        pltpu.SemaphoreType.DMA,
        pltpu.SemaphoreType.DMA,
        pltpu.SemaphoreType.DMA,
    ],
    compiler_params=pltpu.CompilerParams(needs_layout_passes=False),
)(_sc_body)


def kernel(id1, W):
    idt = id1.T
    wflat = jnp.pad(W.reshape(-1), (1, L - 2 * EMB - 1)).astype(jnp.float32)
    out_t = _sc_lookup(idt, wflat)
    return jnp.transpose(out_t, (2, 1, 0))


# speedup vs baseline: 236.9147x; 1.1569x over previous
import functools

import jax
import jax.numpy as jnp
from jax import lax
from jax.experimental import pallas as pl
from jax.experimental.pallas import tpu as pltpu
from jax.experimental.pallas import tpu_sc as plsc

NC = 2
NS = 16
NW = NC * NS
L = 16
EMB = 5

N_ROWS, N_COLS = 16384, 200
I_SPAN = N_ROWS // NW
NJT = N_COLS // 8


def _sc_body(idt_hbm, w_hbm, out_hbm, id_v, out_v, w_v,
             in_sem0, in_sem1, out_sem0, out_sem1):
    wid = lax.axis_index("s") * NC + lax.axis_index("c")
    pltpu.sync_copy(w_hbm, w_v)
    p0 = [plsc.load_gather(w_v, [jnp.full((L,), k + 1, jnp.int32)])
          for k in range(EMB)]
    p1 = [plsc.load_gather(w_v, [jnp.full((L,), k + 1 + EMB, jnp.int32)])
          for k in range(EMB)]
    i0 = wid * I_SPAN

    in_sems = [in_sem0, in_sem1]
    out_sems = [out_sem0, out_sem1]
    pend_in = [None, None]
    pend_out = [[], []]

    def start_in(jt):
        b = jt & 1
        pend_in[b] = pltpu.async_copy(
            idt_hbm.at[pl.ds(jt * 8, 8), pl.ds(i0, I_SPAN)],
            id_v.at[b], in_sems[b])

    start_in(0)
    for jt in range(NJT):
        b = jt & 1
        if jt + 1 < NJT:
            start_in(jt + 1)
        pend_in[b].wait()
        for h in pend_out[b]:
            h.wait()
        pend_out[b] = []

        @plsc.parallel_loop(0, (8 * I_SPAN) // L, unroll=4)
        def c_body(t):
            j = t >> 5
            off = (t & 31) * L
            m = id_v[b, j, pl.ds(off, L)] == 0
            for k in range(EMB):
                out_v[b, k, j, pl.ds(off, L)] = jnp.where(m, p0[k], p1[k])

        for k in range(EMB):
            pend_out[b].append(pltpu.async_copy(
                out_v.at[b, k],
                out_hbm.at[k, pl.ds(jt * 8, 8), pl.ds(i0, I_SPAN)],
                out_sems[b]))

    for b in range(2):
        for h in pend_out[b]:
            h.wait()


_mesh = plsc.VectorSubcoreMesh(core_axis_name="c", subcore_axis_name="s")

_sc_lookup = functools.partial(
    pl.kernel,
    mesh=_mesh,
    out_type=jax.ShapeDtypeStruct((EMB, N_COLS, N_ROWS), jnp.float32),
    scratch_types=[
        pltpu.VMEM((2, 8, I_SPAN), jnp.int32),
        pltpu.VMEM((2, EMB, 8, I_SPAN), jnp.float32),
        pltpu.VMEM((L,), jnp.float32),
        pltpu.SemaphoreType.DMA,
        pltpu.SemaphoreType.DMA,
        pltpu.SemaphoreType.DMA,
        pltpu.SemaphoreType.DMA,
    ],
    compiler_params=pltpu.CompilerParams(needs_layout_passes=False),
)(_sc_body)


def kernel(id1, W):
    idt = id1.T
    wflat = jnp.pad(W.reshape(-1), (1, L - 2 * EMB - 1)).astype(jnp.float32)
    out_t = _sc_lookup(idt, wflat)
    return jnp.transpose(out_t, (2, 1, 0))
